# baseline (speedup 1.0000x reference)
import functools

import jax
import jax.numpy as jnp
from jax import lax
from jax.experimental import pallas as pl
from jax.experimental.pallas import tpu as pltpu
from jax.experimental.pallas import tpu_sc as plsc


@functools.lru_cache(maxsize=None)
def _make_gather(n_ids: int, vocab: int, dim: int):
    info = plsc.get_sparse_core_info()
    num_workers = info.num_cores * info.num_subcores
    assert n_ids % (8 * num_workers) == 0
    per_w = n_ids // num_workers
    n_chunks = 4
    assert per_w % n_chunks == 0
    chunk = per_w // n_chunks

    mesh = plsc.VectorSubcoreMesh(core_axis_name="c", subcore_axis_name="s")

    @functools.partial(
        pl.kernel,
        mesh=mesh,
        out_type=jax.ShapeDtypeStruct((n_ids, dim), jnp.float32),
        scratch_types=[
            pltpu.VMEM((per_w,), jnp.int32),
            pltpu.VMEM((n_chunks, chunk, dim), jnp.float32),
            [pltpu.SemaphoreType.DMA] * n_chunks,
            pltpu.SemaphoreType.DMA,
        ],
    )
    def gather_kernel(table_hbm, idx_hbm, out_hbm, idx_v, rows_v, gsems, ssem):
        wid = lax.axis_index("s") * info.num_cores + lax.axis_index("c")
        base = wid * per_w
        pltpu.sync_copy(idx_hbm.at[pl.ds(base, per_w)], idx_v)
        gathers = []
        for c in range(n_chunks):
            gathers.append(
                pltpu.async_copy(
                    table_hbm.at[idx_v.at[pl.ds(c * chunk, chunk)]],
                    rows_v.at[c],
                    gsems[c],
                )
            )
        stores = []
        for c in range(n_chunks):
            gathers[c].wait()
            stores.append(
                pltpu.async_copy(
                    rows_v.at[c],
                    out_hbm.at[pl.ds(base + c * chunk, chunk)],
                    ssem,
                )
            )
        for st in stores:
            st.wait()

    return gather_kernel


def kernel(input_ids, attention_mask, table):
    batch, seq = input_ids.shape
    vocab, dim = table.shape
    n_ids = batch * seq
    flat_ids = input_ids.reshape(n_ids).astype(jnp.int32)
    gather_kernel = _make_gather(n_ids, vocab, dim)
    out = gather_kernel(table, flat_ids)
    return (out.reshape(batch, seq, dim), attention_mask)


# speedup vs baseline: 1.0077x; 1.0077x over previous
import functools

import jax
import jax.numpy as jnp
from jax import lax
from jax.experimental import pallas as pl
from jax.experimental.pallas import tpu as pltpu
from jax.experimental.pallas import tpu_sc as plsc


@functools.lru_cache(maxsize=None)
def _make_gather(batch: int, seq: int, vocab: int, dim: int):
    info = plsc.get_sparse_core_info()
    num_workers = info.num_cores * info.num_subcores
    n_ids = batch * seq
    assert n_ids % (8 * num_workers) == 0
    per_w = n_ids // num_workers
    w_per_b = seq // per_w
    assert seq % per_w == 0

    mesh = plsc.VectorSubcoreMesh(core_axis_name="c", subcore_axis_name="s")

    @functools.partial(
        pl.kernel,
        mesh=mesh,
        out_type=jax.ShapeDtypeStruct((batch, seq, dim), jnp.float32),
        scratch_types=[
            pltpu.VMEM((per_w,), jnp.int32),
            pltpu.VMEM((per_w, dim), jnp.float32),
            pltpu.SemaphoreType.DMA,
        ],
    )
    def gather_kernel(table_hbm, idx_hbm, out_hbm, idx_v, rows_v, sem):
        wid = lax.axis_index("s") * info.num_cores + lax.axis_index("c")
        b = wid // w_per_b
        col = (wid % w_per_b) * per_w
        pltpu.sync_copy(idx_hbm.at[b, pl.ds(col, per_w)], idx_v)
        pltpu.async_copy(table_hbm.at[idx_v], rows_v, sem).wait()
        pltpu.sync_copy(rows_v, out_hbm.at[b, pl.ds(col, per_w)])

    return gather_kernel


def kernel(input_ids, attention_mask, table):
    batch, seq = input_ids.shape
    vocab, dim = table.shape
    gather_kernel = _make_gather(batch, seq, vocab, dim)
    out = gather_kernel(table, input_ids.astype(jnp.int32))
    return (out, attention_mask)


# speedup vs baseline: 1.0292x; 1.0214x over previous
import functools

import jax
import jax.numpy as jnp
from jax import lax
from jax.experimental import pallas as pl
from jax.experimental.pallas import tpu as pltpu
from jax.experimental.pallas import tpu_sc as plsc


@functools.lru_cache(maxsize=None)
def _make_gather(batch: int, seq: int, vocab: int, dim: int):
    info = plsc.get_sparse_core_info()
    num_workers = info.num_cores * info.num_subcores
    n_ids = batch * seq
    assert n_ids % (8 * num_workers) == 0
    per_w = n_ids // num_workers
    w_per_b = seq // per_w
    assert seq % per_w == 0

    mesh = plsc.VectorSubcoreMesh(core_axis_name="c", subcore_axis_name="s")

    @functools.partial(
        pl.kernel,
        mesh=mesh,
        out_type=(
            jax.ShapeDtypeStruct((batch, seq, dim), jnp.float32),
            jax.ShapeDtypeStruct((batch, seq), jnp.int32),
        ),
        scratch_types=[
            pltpu.VMEM((per_w,), jnp.int32),
            pltpu.VMEM((per_w, dim), jnp.float32),
            pltpu.SemaphoreType.DMA,
            pltpu.SemaphoreType.DMA,
        ],
    )
    def gather_kernel(table_hbm, idx_hbm, mask_hbm, out_hbm, mask_out_hbm,
                      idx_v, rows_v, sem, msem):
        wid = lax.axis_index("s") * info.num_cores + lax.axis_index("c")
        b = wid // w_per_b
        col = (wid % w_per_b) * per_w
        mask_cp = pltpu.async_copy(
            mask_hbm.at[b, pl.ds(col, per_w)],
            mask_out_hbm.at[b, pl.ds(col, per_w)],
            msem,
        )
        pltpu.sync_copy(idx_hbm.at[b, pl.ds(col, per_w)], idx_v)
        pltpu.async_copy(table_hbm.at[idx_v], rows_v, sem).wait()
        pltpu.sync_copy(rows_v, out_hbm.at[b, pl.ds(col, per_w)])
        mask_cp.wait()

    return gather_kernel


def kernel(input_ids, attention_mask, table):
    batch, seq = input_ids.shape
    vocab, dim = table.shape
    gather_kernel = _make_gather(batch, seq, vocab, dim)
    out, mask_out = gather_kernel(
        table, input_ids.astype(jnp.int32), attention_mask
    )
    return (out, mask_out)


# speedup vs baseline: 1.0301x; 1.0008x over previous
import functools

import jax
import jax.numpy as jnp
from jax import lax
from jax.experimental import pallas as pl
from jax.experimental.pallas import tpu as pltpu
from jax.experimental.pallas import tpu_sc as plsc


@functools.lru_cache(maxsize=None)
def _make_gather(batch: int, seq: int, vocab: int, dim: int, mask_dtype):
    info = plsc.get_sparse_core_info()
    num_workers = info.num_cores * info.num_subcores
    n_ids = batch * seq
    assert n_ids % (8 * num_workers) == 0
    per_w = n_ids // num_workers
    w_per_b = seq // per_w
    assert seq % per_w == 0

    mesh = plsc.VectorSubcoreMesh(core_axis_name="c", subcore_axis_name="s")

    @functools.partial(
        pl.kernel,
        mesh=mesh,
        out_type=(
            jax.ShapeDtypeStruct((batch, seq, dim), jnp.float32),
            jax.ShapeDtypeStruct((batch, seq), mask_dtype),
        ),
        scratch_types=[
            pltpu.VMEM((per_w,), jnp.int32),
            pltpu.VMEM((per_w, dim), jnp.float32),
            pltpu.SemaphoreType.DMA,
            pltpu.SemaphoreType.DMA,
        ],
    )
    def gather_kernel(table_hbm, idx_hbm, mask_hbm, out_hbm, mask_out_hbm,
                      idx_v, rows_v, sem, msem):
        wid = lax.axis_index("s") * info.num_cores + lax.axis_index("c")
        b = wid // w_per_b
        col = (wid % w_per_b) * per_w
        pltpu.sync_copy(idx_hbm.at[b, pl.ds(col, per_w)], idx_v)
        gather = pltpu.async_copy(table_hbm.at[idx_v], rows_v, sem)
        mask_cp = pltpu.async_copy(
            mask_hbm.at[b, pl.ds(col, per_w)],
            mask_out_hbm.at[b, pl.ds(col, per_w)],
            msem,
        )
        gather.wait()
        pltpu.sync_copy(rows_v, out_hbm.at[b, pl.ds(col, per_w)])
        mask_cp.wait()

    return gather_kernel


def kernel(input_ids, attention_mask, table):
    batch, seq = input_ids.shape
    vocab, dim = table.shape
    gather_kernel = _make_gather(
        batch, seq, vocab, dim, jnp.dtype(attention_mask.dtype)
    )
    out, mask_out = gather_kernel(
        table, input_ids.astype(jnp.int32), attention_mask
    )
    return (out, mask_out)
